# baseline (device time: 164118 ns/iter reference)
import jax
import jax.numpy as jnp
from jax import lax
from jax.experimental import pallas as pl
from jax.experimental.pallas import tpu as pltpu

N_DEV = 4
GELU_C = 0.7978845608028654


def kernel(x, w_mat):
    x = x.astype(jnp.bfloat16)
    k_glob, k_loc = x.shape
    _, n_glob = w_mat.shape
    m_loc = k_glob // N_DEV
    n_b = 8
    n_blk = n_glob // n_b
    assert m_loc == k_loc

    s_order = [0, 1, 3, 2]
    steps = [(s, nb) for s in s_order for nb in range(n_b)]

    def body(x_ref, w_hbm, out_ref, recv_ref, w_vmem, xf32_ref,
             send_sems, recv_sems, copy_sems):
        i = lax.axis_index("i")

        barrier = pltpu.get_barrier_semaphore()
        for d in range(1, N_DEV):
            pl.semaphore_signal(
                barrier, inc=1,
                device_id=((i + d) % N_DEV,),
                device_id_type=pl.DeviceIdType.MESH)
        pl.semaphore_wait(barrier, N_DEV - 1)

        sends = []
        for d in range(1, N_DEV):
            t = (i + d) % N_DEV
            rdma = pltpu.make_async_remote_copy(
                src_ref=x_ref.at[pl.ds(t * m_loc, m_loc)],
                dst_ref=recv_ref.at[3 - d],
                send_sem=send_sems.at[d - 1],
                recv_sem=recv_sems.at[3 - d],
                device_id=(t,),
                device_id_type=pl.DeviceIdType.MESH)
            rdma.start()
            sends.append(rdma)

        def w_copy(step_idx, slot):
            s, nb = steps[step_idx]
            kb = (i + s) % N_DEV
            return pltpu.make_async_copy(
                w_hbm.at[pl.ds(kb * m_loc, m_loc),
                         pl.ds(nb * n_blk, n_blk)],
                w_vmem.at[slot],
                copy_sems.at[slot])

        w_copy(0, 0).start()

        for step_idx, (s, nb) in enumerate(steps):
            slot = step_idx % 2
            if step_idx + 1 < len(steps):
                w_copy(step_idx + 1, (step_idx + 1) % 2).start()
            if s != 0 and nb == 0:
                recv_wait = pltpu.make_async_remote_copy(
                    src_ref=recv_ref.at[s - 1],
                    dst_ref=recv_ref.at[s - 1],
                    send_sem=send_sems.at[0],
                    recv_sem=recv_sems.at[s - 1],
                    device_id=(i,),
                    device_id_type=pl.DeviceIdType.MESH)
                recv_wait.wait_recv()
            if nb == 0:
                if s == 0:
                    xblk = x_ref[pl.ds(i * m_loc, m_loc), :]
                else:
                    xblk = recv_ref[s - 1]
                xf32_ref[...] = xblk.astype(jnp.float32)
            w_copy(step_idx, slot).wait()
            partial = jnp.dot(xf32_ref[...], w_vmem[slot],
                              preferred_element_type=jnp.float32)
            nsl = slice(nb * n_blk, (nb + 1) * n_blk)
            if step_idx < n_b:
                out_ref[:, nsl] = partial
            else:
                acc = out_ref[:, nsl] + partial
                if step_idx >= 3 * n_b:
                    acc = 0.5 * acc * (1.0 + jnp.tanh(
                        GELU_C * (acc + 0.044715 * acc * acc * acc)))
                out_ref[:, nsl] = acc

        for rdma in sends:
            rdma.wait_send()

    return pl.pallas_call(
        body,
        out_shape=jax.ShapeDtypeStruct((m_loc, n_glob), jnp.float32),
        in_specs=[
            pl.BlockSpec(memory_space=pltpu.VMEM),
            pl.BlockSpec(memory_space=pl.ANY),
        ],
        out_specs=pl.BlockSpec(memory_space=pltpu.VMEM),
        scratch_shapes=[
            pltpu.VMEM((N_DEV - 1, m_loc, k_loc), x.dtype),
            pltpu.VMEM((2, m_loc, n_blk), w_mat.dtype),
            pltpu.VMEM((m_loc, k_loc), jnp.float32),
            pltpu.SemaphoreType.DMA((N_DEV - 1,)),
            pltpu.SemaphoreType.DMA((N_DEV - 1,)),
            pltpu.SemaphoreType.DMA((2,)),
        ],
        compiler_params=pltpu.CompilerParams(
            collective_id=0,
            vmem_limit_bytes=64 * 1024 * 1024,
        ),
    )(x, w_mat)


# device time: 161000 ns/iter; 1.0194x vs baseline; 1.0194x over previous
import jax
import jax.numpy as jnp
from jax import lax
from jax.experimental import pallas as pl
from jax.experimental.pallas import tpu as pltpu

N_DEV = 4
GELU_C = 0.7978845608028654
W_SLOTS = 4


def kernel(x, w_mat):
    x = x.astype(jnp.bfloat16)
    k_glob, k_loc = x.shape
    _, n_glob = w_mat.shape
    m_loc = k_glob // N_DEV
    n_b = 8
    n_blk = n_glob // n_b
    assert m_loc == k_loc

    s_order = [0, 1, 3, 2]
    steps = [(s, nb) for s in s_order for nb in range(n_b)]

    def body(x_hbm, w_hbm, out_ref, recv_ref, xloc_ref, xf32_ref,
             w_vmem, send_sems, recv_sems, copy_sems, xloc_sem):
        i = lax.axis_index("i")

        xloc_copy = pltpu.make_async_copy(
            x_hbm.at[pl.ds(i * m_loc, m_loc)], xloc_ref, xloc_sem)
        xloc_copy.start()

        barrier = pltpu.get_barrier_semaphore()
        for d in range(1, N_DEV):
            pl.semaphore_signal(
                barrier, inc=1,
                device_id=((i + d) % N_DEV,),
                device_id_type=pl.DeviceIdType.MESH)
        pl.semaphore_wait(barrier, N_DEV - 1)

        sends = []
        for d in range(1, N_DEV):
            t = (i + d) % N_DEV
            rdma = pltpu.make_async_remote_copy(
                src_ref=x_hbm.at[pl.ds(t * m_loc, m_loc)],
                dst_ref=recv_ref.at[3 - d],
                send_sem=send_sems.at[d - 1],
                recv_sem=recv_sems.at[3 - d],
                device_id=(t,),
                device_id_type=pl.DeviceIdType.MESH)
            rdma.start()
            sends.append(rdma)

        def w_copy(step_idx):
            s, nb = steps[step_idx]
            kb = (i + s) % N_DEV
            slot = step_idx % W_SLOTS
            return pltpu.make_async_copy(
                w_hbm.at[pl.ds(kb * m_loc, m_loc),
                         pl.ds(nb * n_blk, n_blk)],
                w_vmem.at[slot],
                copy_sems.at[slot])

        for idx in range(W_SLOTS):
            w_copy(idx).start()

        xloc_copy.wait()

        for step_idx, (s, nb) in enumerate(steps):
            slot = step_idx % W_SLOTS
            if s != 0 and nb == 0:
                recv_wait = pltpu.make_async_remote_copy(
                    src_ref=recv_ref.at[s - 1],
                    dst_ref=recv_ref.at[s - 1],
                    send_sem=send_sems.at[0],
                    recv_sem=recv_sems.at[s - 1],
                    device_id=(i,),
                    device_id_type=pl.DeviceIdType.MESH)
                recv_wait.wait_recv()
            if nb == 0:
                xblk = xloc_ref[...] if s == 0 else recv_ref[s - 1]
                xf32_ref[...] = xblk.astype(jnp.float32)
            w_copy(step_idx).wait()
            partial = jnp.dot(xf32_ref[...], w_vmem[slot],
                              preferred_element_type=jnp.float32)
            nsl = slice(nb * n_blk, (nb + 1) * n_blk)
            if step_idx < n_b:
                out_ref[:, nsl] = partial
            else:
                acc = out_ref[:, nsl] + partial
                if step_idx >= 3 * n_b:
                    acc = 0.5 * acc * (1.0 + jnp.tanh(
                        GELU_C * (acc + 0.044715 * acc * acc * acc)))
                out_ref[:, nsl] = acc
            if step_idx + W_SLOTS < len(steps):
                w_copy(step_idx + W_SLOTS).start()

        for rdma in sends:
            rdma.wait_send()

    return pl.pallas_call(
        body,
        out_shape=jax.ShapeDtypeStruct((m_loc, n_glob), jnp.float32),
        in_specs=[
            pl.BlockSpec(memory_space=pl.ANY),
            pl.BlockSpec(memory_space=pl.ANY),
        ],
        out_specs=pl.BlockSpec(memory_space=pltpu.VMEM),
        scratch_shapes=[
            pltpu.VMEM((N_DEV - 1, m_loc, k_loc), x.dtype),
            pltpu.VMEM((m_loc, k_loc), x.dtype),
            pltpu.VMEM((m_loc, k_loc), jnp.float32),
            pltpu.VMEM((W_SLOTS, m_loc, n_blk), w_mat.dtype),
            pltpu.SemaphoreType.DMA((N_DEV - 1,)),
            pltpu.SemaphoreType.DMA((N_DEV - 1,)),
            pltpu.SemaphoreType.DMA((W_SLOTS,)),
            pltpu.SemaphoreType.DMA,
        ],
        compiler_params=pltpu.CompilerParams(
            collective_id=0,
            vmem_limit_bytes=64 * 1024 * 1024,
        ),
    )(x, w_mat)
